# baseline (speedup 1.0000x reference)
import functools

import jax
import jax.numpy as jnp
from jax.experimental import pallas as pl
from jax.experimental.pallas import tpu as pltpu

N_TOKENS = 32768
D = 768
E = 8
K = 2
BT = 2048
EPAD = 128


def _router_block(x_ref, wt_ref, idx_ref, topk_ref, probs_ref):
    x = x_ref[...]
    wt = wt_ref[...]
    logits = jnp.dot(x, wt, preferred_element_type=jnp.float32)

    col = jax.lax.broadcasted_iota(jnp.int32, (BT, EPAD), 1)
    valid = col < E
    neg = jnp.float32(-jnp.inf)
    logits = jnp.where(valid, logits, neg)

    m = jnp.max(logits, axis=1, keepdims=True)
    ex = jnp.where(valid, jnp.exp(logits - m), 0.0)
    denom = jnp.sum(ex, axis=1, keepdims=True)
    probs = ex / denom

    p1 = jnp.max(probs, axis=1, keepdims=True)
    big = jnp.int32(EPAD)
    i1 = jnp.min(jnp.where((probs == p1) & valid, col, big), axis=1, keepdims=True)
    rest = jnp.where((col != i1) & valid, probs, -1.0)
    p2 = jnp.max(rest, axis=1, keepdims=True)
    i2 = jnp.min(jnp.where(rest == p2, col, big), axis=1, keepdims=True)

    rn = 1.0 / (p1 + p2 + 1e-9)

    probs_ref[...] = probs[:, :E]
    idx_ref[...] = jnp.concatenate([i1, i2], axis=1)
    topk_ref[...] = jnp.concatenate([p1 * rn, p2 * rn], axis=1)


@jax.jit
def kernel(x, W_gate, W_noisy):
    wt = jnp.zeros((D, EPAD), jnp.float32).at[:, :E].set(W_gate.T)
    grid = (N_TOKENS // BT,)
    out_shapes = (
        jax.ShapeDtypeStruct((N_TOKENS, K), jnp.int32),
        jax.ShapeDtypeStruct((N_TOKENS, K), jnp.float32),
        jax.ShapeDtypeStruct((N_TOKENS, E), jnp.float32),
    )
    topk_idx, topk_probs, probs = pl.pallas_call(
        _router_block,
        grid=grid,
        in_specs=[
            pl.BlockSpec((BT, D), lambda i: (i, 0)),
            pl.BlockSpec((D, EPAD), lambda i: (0, 0)),
        ],
        out_specs=(
            pl.BlockSpec((BT, K), lambda i: (i, 0)),
            pl.BlockSpec((BT, K), lambda i: (i, 0)),
            pl.BlockSpec((BT, E), lambda i: (i, 0)),
        ),
        out_shape=out_shapes,
    )(x, wt)
    return topk_idx, topk_probs, probs


# speedup vs baseline: 1.1258x; 1.1258x over previous
import jax
import jax.numpy as jnp
from jax.experimental import pallas as pl

N_TOKENS = 32768
D = 768
E = 8
K = 2
BT = 2048


def _router_block(x_ref, w_ref, idx_ref, topk_ref, probs_ref):
    x = x_ref[...]
    w = w_ref[...]
    logits_t = jax.lax.dot_general(
        w, x, (((1,), (1,)), ((), ())), preferred_element_type=jnp.float32)

    m = jnp.max(logits_t, axis=0, keepdims=True)
    ex = jnp.exp(logits_t - m)
    denom = jnp.sum(ex, axis=0, keepdims=True)
    probs_t = ex / denom

    row = jax.lax.broadcasted_iota(jnp.int32, (E, BT), 0)
    big = jnp.int32(E)
    p1 = jnp.max(probs_t, axis=0, keepdims=True)
    i1 = jnp.min(jnp.where(probs_t == p1, row, big), axis=0, keepdims=True)
    rest = jnp.where(row != i1, probs_t, -1.0)
    p2 = jnp.max(rest, axis=0, keepdims=True)
    i2 = jnp.min(jnp.where(rest == p2, row, big), axis=0, keepdims=True)

    rn = 1.0 / (p1 + p2 + 1e-9)

    probs_ref[...] = probs_t.T
    idx_ref[...] = jnp.concatenate([i1, i2], axis=0).T
    topk_ref[...] = jnp.concatenate([p1 * rn, p2 * rn], axis=0).T


@jax.jit
def kernel(x, W_gate, W_noisy):
    grid = (N_TOKENS // BT,)
    out_shapes = (
        jax.ShapeDtypeStruct((N_TOKENS, K), jnp.int32),
        jax.ShapeDtypeStruct((N_TOKENS, K), jnp.float32),
        jax.ShapeDtypeStruct((N_TOKENS, E), jnp.float32),
    )
    topk_idx, topk_probs, probs = pl.pallas_call(
        _router_block,
        grid=grid,
        in_specs=[
            pl.BlockSpec((BT, D), lambda i: (i, 0)),
            pl.BlockSpec((E, D), lambda i: (0, 0)),
        ],
        out_specs=(
            pl.BlockSpec((BT, K), lambda i: (i, 0)),
            pl.BlockSpec((BT, K), lambda i: (i, 0)),
            pl.BlockSpec((BT, E), lambda i: (i, 0)),
        ),
        out_shape=out_shapes,
    )(x, W_gate)
    return topk_idx, topk_probs, probs
